# baseline (speedup 1.0000x reference)
import functools

import jax
import jax.numpy as jnp
from jax import lax
from jax.experimental import pallas as pl
from jax.experimental.pallas import tpu as pltpu
from jax.experimental.pallas import tpu_sc as plsc

_B1, _B2, _S, _D = 16, 16, 4096, 64
_NW = 32
_SLABS = 8
_CH = 256
_CPS = _S // _CH
_NC = _SLABS * _CPS


def _sc_body(prev_hbm, cur_hbm, pos_hbm, out_hbm,
             buf0, buf1, curv, posv, r0, r1, w0, w1):
    cid = lax.axis_index("c")
    sid = lax.axis_index("s")
    wid = sid * 2 + cid
    bufs = (buf0, buf1)
    rs = (r0, r1)
    ws = (w0, w1)

    pltpu.sync_copy(pos_hbm, posv)
    pltpu.sync_copy(cur_hbm.at[pl.ds(wid * _SLABS, _SLABS)], curv)
    p = posv[...][0]

    def slab_of(i):
        s = wid * _SLABS + i // _CPS
        return s // _B2, lax.rem(s, _B2), lax.rem(i, _CPS)

    def start_read(i, b):
        b1, b2, c = slab_of(i)
        pltpu.async_copy(
            prev_hbm.at[b1, b2, pl.ds(c * _CH, _CH)], bufs[b], rs[b])

    start_read(0, 0)
    start_read(1, 1)

    def step(g, _):
        for b in range(2):
            i = 2 * g + b
            b1, b2, c = slab_of(i)
            pltpu.make_async_copy(
                prev_hbm.at[b1, b2, pl.ds(c * _CH, _CH)],
                bufs[b], rs[b]).wait()
            wcp = pltpu.make_async_copy(
                bufs[b], out_hbm.at[b1, b2, pl.ds(c * _CH, _CH)], ws[b])
            wcp.start()
            wcp.wait()
            nxt = i + 2

            @pl.when(nxt < _NC)
            def _():
                start_read(nxt, b)

        return 0

    lax.fori_loop(0, _NC // 2, step, 0)

    for j in range(_SLABS):
        s = wid * _SLABS + j
        b1 = s // _B2
        b2 = lax.rem(s, _B2)
        pltpu.sync_copy(
            curv.at[pl.ds(j, 1)],
            out_hbm.at[b1, b2, pl.ds(p, 1)])


def kernel(prev, cur, dim, idx):
    B1, B2, S, D = prev.shape
    pos = (idx - (dim - 1)).astype(jnp.int32)
    pos16 = jnp.broadcast_to(pos, (16,))
    c2 = cur.reshape(B1 * B2, D)
    fn = functools.partial(
        pl.kernel,
        mesh=plsc.VectorSubcoreMesh(core_axis_name="c", subcore_axis_name="s"),
        out_type=jax.ShapeDtypeStruct(prev.shape, prev.dtype),
        scratch_types=[
            pltpu.VMEM((_CH, _D), jnp.float32),
            pltpu.VMEM((_CH, _D), jnp.float32),
            pltpu.VMEM((_SLABS, _D), jnp.float32),
            pltpu.VMEM((16,), jnp.int32),
            pltpu.SemaphoreType.DMA,
            pltpu.SemaphoreType.DMA,
            pltpu.SemaphoreType.DMA,
            pltpu.SemaphoreType.DMA,
        ],
    )(_sc_body)
    return fn(prev, c2, pos16)


# speedup vs baseline: 1.5337x; 1.5337x over previous
import jax
import jax.numpy as jnp
from jax.experimental import pallas as pl
from jax.experimental.pallas import tpu as pltpu


def _body(pos_ref, prev_ref, cur_ref, out_ref, sem):
    del prev_ref
    p = pos_ref[0]
    cp = pltpu.make_async_copy(
        cur_ref, out_ref.at[:, :, pl.ds(p, 1), :], sem)
    cp.start()
    cp.wait()


def kernel(prev, cur, dim, idx):
    pos = (idx - (dim - 1)).astype(jnp.int32)
    out = pl.pallas_call(
        _body,
        grid_spec=pltpu.PrefetchScalarGridSpec(
            num_scalar_prefetch=1,
            grid=(1,),
            in_specs=[
                pl.BlockSpec(memory_space=pl.ANY),
                pl.BlockSpec(memory_space=pl.ANY),
            ],
            out_specs=pl.BlockSpec(memory_space=pl.ANY),
            scratch_shapes=[pltpu.SemaphoreType.DMA],
        ),
        out_shape=jax.ShapeDtypeStruct(prev.shape, prev.dtype),
        input_output_aliases={1: 0},
    )(pos, prev, cur)
    return out


# speedup vs baseline: 1.9257x; 1.2556x over previous
import jax
import jax.numpy as jnp
from jax.experimental import pallas as pl
from jax.experimental.pallas import tpu as pltpu


def _body(pos_ref, prev_ref, cur_ref, out_ref, sem):
    del prev_ref
    p = pos_ref[0]
    cp = pltpu.make_async_copy(
        cur_ref, out_ref.at[:, :, pl.ds(p, 1), :], sem)
    cp.start()
    cp.wait()


def kernel(prev, cur, dim, idx):
    pos = (idx - (dim - 1)).astype(jnp.int32)
    j0 = jnp.minimum(idx[0].astype(jnp.int32), jnp.int32(0))
    tmp = jax.lax.dynamic_update_slice(
        prev, prev[0:1, 0:1, 0:1, 0:1], (j0, 0, 0, 0))
    out = pl.pallas_call(
        _body,
        grid_spec=pltpu.PrefetchScalarGridSpec(
            num_scalar_prefetch=1,
            grid=(1,),
            in_specs=[
                pl.BlockSpec(memory_space=pl.ANY),
                pl.BlockSpec(memory_space=pl.ANY),
            ],
            out_specs=pl.BlockSpec(memory_space=pl.ANY),
            scratch_shapes=[pltpu.SemaphoreType.DMA],
        ),
        out_shape=jax.ShapeDtypeStruct(prev.shape, prev.dtype),
        input_output_aliases={1: 0},
    )(pos, tmp, cur)
    return out
